# baseline (speedup 1.0000x reference)
import functools

import jax
import jax.numpy as jnp
from jax.experimental import pallas as pl

_B = 4096
_D = 256
_K = 8192
_TEMP = 0.05

_BB = 256
_BC = 512


def _loss_kernel(x_ref, t_ref, f_ref, out_ref):
    x = x_ref[:, :]
    norm = jnp.sqrt(jnp.sum(x * x, axis=1, keepdims=True))
    xn = x / (jnp.maximum(norm, 1e-12) * _TEMP)

    t = t_ref[0, 0, :]

    n_chunks = (2 * _K) // _BC

    def body(ci, carry):
        sm, sh, tm, th = carry
        chunk = f_ref[pl.ds(ci * _BC, _BC), :]
        logits = jax.lax.dot_general(
            xn, chunk,
            dimension_numbers=(((1,), (1,)), ((), ())),
            preferred_element_type=jnp.float32,
            precision=jax.lax.Precision.HIGHEST,
        )
        e = jnp.exp(logits)
        s = jnp.sum(e, axis=1)
        cols = ci * _BC + jax.lax.broadcasted_iota(jnp.int32, (_BB, _BC), 1)
        mmask = cols == t[:, None]
        hmask = cols == t[:, None] + _K
        tm = tm + jnp.sum(jnp.where(mmask, logits, 0.0), axis=1)
        th = th + jnp.sum(jnp.where(hmask, logits, 0.0), axis=1)
        is_mean = ci < (n_chunks // 2)
        sm = jnp.where(is_mean, sm + s, sm)
        sh = jnp.where(is_mean, sh, sh + s)
        return sm, sh, tm, th

    z = jnp.zeros((_BB,), jnp.float32)
    sm, sh, tm, th = jax.lax.fori_loop(0, n_chunks, body, (z, z, z, z))

    nll = 0.5 * ((jnp.log(sm) - tm) + (jnp.log(sh) - th))
    out_ref[:, 0] = nll


@jax.jit
def _run(inputs, targets, features):
    nb = _B // _BB
    t3 = targets.astype(jnp.int32).reshape(nb, 1, _BB)
    nll = pl.pallas_call(
        _loss_kernel,
        grid=(nb,),
        in_specs=[
            pl.BlockSpec((_BB, _D), lambda b: (b, 0)),
            pl.BlockSpec((1, 1, _BB), lambda b: (b, 0, 0)),
            pl.BlockSpec((2 * _K, _D), lambda b: (0, 0)),
        ],
        out_specs=pl.BlockSpec((_BB, 1), lambda b: (b, 0)),
        out_shape=jax.ShapeDtypeStruct((_B, 1), jnp.float32),
    )(inputs, t3, features)
    return jnp.mean(nll)


def kernel(inputs, targets, features):
    return _run(inputs, targets, features)


# speedup vs baseline: 1.7220x; 1.7220x over previous
import functools

import jax
import jax.numpy as jnp
from jax.experimental import pallas as pl

_B = 4096
_D = 256
_K = 8192
_TEMP = 0.05

_BB = 256
_BC = 512


def _loss_kernel(x_ref, t_ref, f_ref, out_ref):
    x = x_ref[:, :]
    norm = jnp.sqrt(jnp.sum(x * x, axis=1, keepdims=True))
    xn = x / (jnp.maximum(norm, 1e-12) * _TEMP)

    t = t_ref[0, 0, :]

    n_chunks = (2 * _K) // _BC

    def body(ci, carry):
        sm, sh, tm, th = carry
        chunk = f_ref[pl.ds(ci * _BC, _BC), :]
        logits = jax.lax.dot_general(
            xn, chunk,
            dimension_numbers=(((1,), (1,)), ((), ())),
            preferred_element_type=jnp.float32,
            precision=jax.lax.Precision.DEFAULT,
        )
        e = jnp.exp(logits)
        s = jnp.sum(e, axis=1)
        cols = ci * _BC + jax.lax.broadcasted_iota(jnp.int32, (_BB, _BC), 1)
        mmask = cols == t[:, None]
        hmask = cols == t[:, None] + _K
        tm = tm + jnp.sum(jnp.where(mmask, logits, 0.0), axis=1)
        th = th + jnp.sum(jnp.where(hmask, logits, 0.0), axis=1)
        is_mean = ci < (n_chunks // 2)
        sm = jnp.where(is_mean, sm + s, sm)
        sh = jnp.where(is_mean, sh, sh + s)
        return sm, sh, tm, th

    z = jnp.zeros((_BB,), jnp.float32)
    sm, sh, tm, th = jax.lax.fori_loop(0, n_chunks, body, (z, z, z, z))

    nll = 0.5 * ((jnp.log(sm) - tm) + (jnp.log(sh) - th))
    out_ref[:, 0] = nll


@jax.jit
def _run(inputs, targets, features):
    nb = _B // _BB
    t3 = targets.astype(jnp.int32).reshape(nb, 1, _BB)
    nll = pl.pallas_call(
        _loss_kernel,
        grid=(nb,),
        in_specs=[
            pl.BlockSpec((_BB, _D), lambda b: (b, 0)),
            pl.BlockSpec((1, 1, _BB), lambda b: (b, 0, 0)),
            pl.BlockSpec((2 * _K, _D), lambda b: (0, 0)),
        ],
        out_specs=pl.BlockSpec((_BB, 1), lambda b: (b, 0)),
        out_shape=jax.ShapeDtypeStruct((_B, 1), jnp.float32),
    )(inputs, t3, features)
    return jnp.mean(nll)


def kernel(inputs, targets, features):
    return _run(inputs, targets, features)


# speedup vs baseline: 1.9242x; 1.1174x over previous
import functools

import jax
import jax.numpy as jnp
from jax.experimental import pallas as pl

_B = 4096
_D = 256
_K = 8192
_TEMP = 0.05

_BB = 256
_BC = 512


def _loss_kernel(x_ref, t_ref, f_ref, out_ref):
    x = x_ref[:, :]
    norm = jnp.sqrt(jnp.sum(x * x, axis=1, keepdims=True))
    xn = x / (jnp.maximum(norm, 1e-12) * _TEMP)

    t = t_ref[0, 0, :]

    n_chunks = (2 * _K) // _BC
    half = n_chunks // 2
    liota = jax.lax.broadcasted_iota(jnp.int32, (_BB, _BC), 1)

    def body(ci, carry):
        sm, sh, tm, th = carry
        chunk = f_ref[pl.ds(ci * _BC, _BC), :]
        logits = jax.lax.dot_general(
            xn, chunk,
            dimension_numbers=(((1,), (1,)), ((), ())),
            preferred_element_type=jnp.float32,
            precision=jax.lax.Precision.DEFAULT,
        )
        e = jnp.exp(logits)
        s = jnp.sum(e, axis=1)
        is_mean = ci < half
        tloc = t - jnp.where(is_mean, ci * _BC, (ci - half) * _BC)
        contrib = jnp.sum(jnp.where(liota == tloc[:, None], logits, 0.0), axis=1)
        tm = tm + jnp.where(is_mean, contrib, 0.0)
        th = th + jnp.where(is_mean, 0.0, contrib)
        sm = jnp.where(is_mean, sm + s, sm)
        sh = jnp.where(is_mean, sh, sh + s)
        return sm, sh, tm, th

    z = jnp.zeros((_BB,), jnp.float32)
    sm, sh, tm, th = jax.lax.fori_loop(0, n_chunks, body, (z, z, z, z))

    nll = 0.5 * ((jnp.log(sm) - tm) + (jnp.log(sh) - th))
    out_ref[:, 0] = nll


@jax.jit
def _run(inputs, targets, features):
    nb = _B // _BB
    t3 = targets.astype(jnp.int32).reshape(nb, 1, _BB)
    nll = pl.pallas_call(
        _loss_kernel,
        grid=(nb,),
        in_specs=[
            pl.BlockSpec((_BB, _D), lambda b: (b, 0)),
            pl.BlockSpec((1, 1, _BB), lambda b: (b, 0, 0)),
            pl.BlockSpec((2 * _K, _D), lambda b: (0, 0)),
        ],
        out_specs=pl.BlockSpec((_BB, 1), lambda b: (b, 0)),
        out_shape=jax.ShapeDtypeStruct((_B, 1), jnp.float32),
    )(inputs, t3, features)
    return jnp.mean(nll)


def kernel(inputs, targets, features):
    return _run(inputs, targets, features)


# speedup vs baseline: 3.0775x; 1.5994x over previous
import functools

import jax
import jax.numpy as jnp
from jax.experimental import pallas as pl

_B = 4096
_D = 256
_K = 8192
_TEMP = 0.05

_BB = 256
_BC = 1024


def _loss_kernel(x_ref, t_ref, f_ref, out_ref):
    x = x_ref[:, :]
    norm = jnp.sqrt(jnp.sum(x * x, axis=1, keepdims=True))
    xn = x * (1.4426950408889634 / (jnp.maximum(norm, 1e-12) * _TEMP))

    t = t_ref[0, 0, :]

    n_chunks = (2 * _K) // _BC
    half = n_chunks // 2
    liota = jax.lax.broadcasted_iota(jnp.int32, (_BB, _BC), 1)

    def body(ci, carry):
        sm, sh, tm, th = carry
        chunk = f_ref[pl.ds(ci * _BC, _BC), :]
        logits = jax.lax.dot_general(
            xn, chunk,
            dimension_numbers=(((1,), (1,)), ((), ())),
            preferred_element_type=jnp.float32,
            precision=jax.lax.Precision.DEFAULT,
        )
        e = jnp.exp2(logits)
        s = jnp.sum(e, axis=1)
        is_mean = ci < half
        tloc = t - jnp.where(is_mean, ci * _BC, (ci - half) * _BC)
        contrib = jnp.sum(jnp.where(liota == tloc[:, None], logits, 0.0), axis=1)
        tm = tm + jnp.where(is_mean, contrib, 0.0)
        th = th + jnp.where(is_mean, 0.0, contrib)
        sm = jnp.where(is_mean, sm + s, sm)
        sh = jnp.where(is_mean, sh, sh + s)
        return sm, sh, tm, th

    z = jnp.zeros((_BB,), jnp.float32)
    sm, sh, tm, th = jax.lax.fori_loop(0, n_chunks, body, (z, z, z, z))

    ln2 = 0.6931471805599453
    nll = (0.5 * ln2) * ((jnp.log2(sm) - tm) + (jnp.log2(sh) - th))
    out_ref[:, 0] = nll


@jax.jit
def _run(inputs, targets, features):
    nb = _B // _BB
    t3 = targets.astype(jnp.int32).reshape(nb, 1, _BB)
    nll = pl.pallas_call(
        _loss_kernel,
        grid=(nb,),
        in_specs=[
            pl.BlockSpec((_BB, _D), lambda b: (b, 0)),
            pl.BlockSpec((1, 1, _BB), lambda b: (b, 0, 0)),
            pl.BlockSpec((2 * _K, _D), lambda b: (0, 0)),
        ],
        out_specs=pl.BlockSpec((_BB, 1), lambda b: (b, 0)),
        out_shape=jax.ShapeDtypeStruct((_B, 1), jnp.float32),
    )(inputs, t3, features)
    return jnp.mean(nll)


def kernel(inputs, targets, features):
    return _run(inputs, targets, features)


# speedup vs baseline: 4.3970x; 1.4288x over previous
import functools

import jax
import jax.numpy as jnp
from jax.experimental import pallas as pl

_B = 4096
_D = 256
_K = 8192
_TEMP = 0.05

_BB = 256
_BC = 2048


def _loss_kernel(x_ref, t_ref, f_ref, out_ref):
    x = x_ref[:, :]
    norm = jnp.sqrt(jnp.sum(x * x, axis=1, keepdims=True))
    xn = x * (1.4426950408889634 / (jnp.maximum(norm, 1e-12) * _TEMP))

    t = t_ref[0, 0, :]

    n_chunks = (2 * _K) // _BC
    half = n_chunks // 2
    liota = jax.lax.broadcasted_iota(jnp.int32, (_BB, _BC), 1)

    def body(ci, carry):
        sm, sh, tm, th = carry
        chunk = f_ref[pl.ds(ci * _BC, _BC), :]
        logits = jax.lax.dot_general(
            xn, chunk,
            dimension_numbers=(((1,), (1,)), ((), ())),
            preferred_element_type=jnp.float32,
            precision=jax.lax.Precision.DEFAULT,
        )
        e = jnp.exp2(logits)
        s = jnp.sum(e, axis=1)
        is_mean = ci < half
        tloc = t - jnp.where(is_mean, ci * _BC, (ci - half) * _BC)
        contrib = jnp.sum(jnp.where(liota == tloc[:, None], logits, 0.0), axis=1)
        tm = tm + jnp.where(is_mean, contrib, 0.0)
        th = th + jnp.where(is_mean, 0.0, contrib)
        sm = jnp.where(is_mean, sm + s, sm)
        sh = jnp.where(is_mean, sh, sh + s)
        return sm, sh, tm, th

    z = jnp.zeros((_BB,), jnp.float32)
    sm, sh, tm, th = jax.lax.fori_loop(0, n_chunks, body, (z, z, z, z))

    ln2 = 0.6931471805599453
    nll = (0.5 * ln2) * ((jnp.log2(sm) - tm) + (jnp.log2(sh) - th))
    out_ref[:, 0] = nll


@jax.jit
def _run(inputs, targets, features):
    nb = _B // _BB
    t3 = targets.astype(jnp.int32).reshape(nb, 1, _BB)
    nll = pl.pallas_call(
        _loss_kernel,
        grid=(nb,),
        in_specs=[
            pl.BlockSpec((_BB, _D), lambda b: (b, 0)),
            pl.BlockSpec((1, 1, _BB), lambda b: (b, 0, 0)),
            pl.BlockSpec((2 * _K, _D), lambda b: (0, 0)),
        ],
        out_specs=pl.BlockSpec((_BB, 1), lambda b: (b, 0)),
        out_shape=jax.ShapeDtypeStruct((_B, 1), jnp.float32),
    )(inputs, t3, features)
    return jnp.mean(nll)


def kernel(inputs, targets, features):
    return _run(inputs, targets, features)


# speedup vs baseline: 5.4122x; 1.2309x over previous
import functools

import jax
import jax.numpy as jnp
from jax.experimental import pallas as pl

_B = 4096
_D = 256
_K = 8192
_TEMP = 0.05

_BB = 256
_BC = 4096


def _loss_kernel(x_ref, t_ref, f_ref, out_ref):
    x = x_ref[:, :]
    norm = jnp.sqrt(jnp.sum(x * x, axis=1, keepdims=True))
    xn = x * (1.4426950408889634 / (jnp.maximum(norm, 1e-12) * _TEMP))

    t = t_ref[0, 0, :]

    n_chunks = (2 * _K) // _BC
    half = n_chunks // 2
    liota = jax.lax.broadcasted_iota(jnp.int32, (_BB, _BC), 1)

    def body(ci, carry):
        sm, sh, tm, th = carry
        chunk = f_ref[pl.ds(ci * _BC, _BC), :]
        logits = jax.lax.dot_general(
            xn, chunk,
            dimension_numbers=(((1,), (1,)), ((), ())),
            preferred_element_type=jnp.float32,
            precision=jax.lax.Precision.DEFAULT,
        )
        e = jnp.exp2(logits)
        s = jnp.sum(e, axis=1)
        is_mean = ci < half
        tloc = t - jnp.where(is_mean, ci * _BC, (ci - half) * _BC)
        contrib = jnp.sum(jnp.where(liota == tloc[:, None], logits, 0.0), axis=1)
        tm = tm + jnp.where(is_mean, contrib, 0.0)
        th = th + jnp.where(is_mean, 0.0, contrib)
        sm = jnp.where(is_mean, sm + s, sm)
        sh = jnp.where(is_mean, sh, sh + s)
        return sm, sh, tm, th

    z = jnp.zeros((_BB,), jnp.float32)
    sm, sh, tm, th = jax.lax.fori_loop(0, n_chunks, body, (z, z, z, z))

    ln2 = 0.6931471805599453
    nll = (0.5 * ln2) * ((jnp.log2(sm) - tm) + (jnp.log2(sh) - th))
    out_ref[:, 0] = nll


@jax.jit
def _run(inputs, targets, features):
    nb = _B // _BB
    t3 = targets.astype(jnp.int32).reshape(nb, 1, _BB)
    nll = pl.pallas_call(
        _loss_kernel,
        grid=(nb,),
        in_specs=[
            pl.BlockSpec((_BB, _D), lambda b: (b, 0)),
            pl.BlockSpec((1, 1, _BB), lambda b: (b, 0, 0)),
            pl.BlockSpec((2 * _K, _D), lambda b: (0, 0)),
        ],
        out_specs=pl.BlockSpec((_BB, 1), lambda b: (b, 0)),
        out_shape=jax.ShapeDtypeStruct((_B, 1), jnp.float32),
    )(inputs, t3, features)
    return jnp.mean(nll)


def kernel(inputs, targets, features):
    return _run(inputs, targets, features)


# speedup vs baseline: 5.7853x; 1.0689x over previous
import functools

import jax
import jax.numpy as jnp
from jax.experimental import pallas as pl

_B = 4096
_D = 256
_K = 8192
_TEMP = 0.05

_BB = 256
_BC = 8192


def _loss_kernel(x_ref, t_ref, f_ref, out_ref):
    x = x_ref[:, :]
    norm = jnp.sqrt(jnp.sum(x * x, axis=1, keepdims=True))
    xn = x * (1.4426950408889634 / (jnp.maximum(norm, 1e-12) * _TEMP))

    t = t_ref[0, 0, :]

    n_chunks = (2 * _K) // _BC
    half = n_chunks // 2
    liota = jax.lax.broadcasted_iota(jnp.int32, (_BB, _BC), 1)

    def body(ci, carry):
        sm, sh, tm, th = carry
        chunk = f_ref[pl.ds(ci * _BC, _BC), :]
        logits = jax.lax.dot_general(
            xn, chunk,
            dimension_numbers=(((1,), (1,)), ((), ())),
            preferred_element_type=jnp.float32,
            precision=jax.lax.Precision.DEFAULT,
        )
        e = jnp.exp2(logits)
        s = jnp.sum(e, axis=1)
        is_mean = ci < half
        tloc = t - jnp.where(is_mean, ci * _BC, (ci - half) * _BC)
        contrib = jnp.sum(jnp.where(liota == tloc[:, None], logits, 0.0), axis=1)
        tm = tm + jnp.where(is_mean, contrib, 0.0)
        th = th + jnp.where(is_mean, 0.0, contrib)
        sm = jnp.where(is_mean, sm + s, sm)
        sh = jnp.where(is_mean, sh, sh + s)
        return sm, sh, tm, th

    z = jnp.zeros((_BB,), jnp.float32)
    sm, sh, tm, th = jax.lax.fori_loop(0, n_chunks, body, (z, z, z, z))

    ln2 = 0.6931471805599453
    nll = (0.5 * ln2) * ((jnp.log2(sm) - tm) + (jnp.log2(sh) - th))
    out_ref[:, 0] = nll


@jax.jit
def _run(inputs, targets, features):
    nb = _B // _BB
    t3 = targets.astype(jnp.int32).reshape(nb, 1, _BB)
    nll = pl.pallas_call(
        _loss_kernel,
        grid=(nb,),
        in_specs=[
            pl.BlockSpec((_BB, _D), lambda b: (b, 0)),
            pl.BlockSpec((1, 1, _BB), lambda b: (b, 0, 0)),
            pl.BlockSpec((2 * _K, _D), lambda b: (0, 0)),
        ],
        out_specs=pl.BlockSpec((_BB, 1), lambda b: (b, 0)),
        out_shape=jax.ShapeDtypeStruct((_B, 1), jnp.float32),
    )(inputs, t3, features)
    return jnp.mean(nll)


def kernel(inputs, targets, features):
    return _run(inputs, targets, features)


# speedup vs baseline: 6.1038x; 1.0550x over previous
import functools

import jax
import jax.numpy as jnp
from jax.experimental import pallas as pl

_B = 4096
_D = 256
_K = 8192
_TEMP = 0.05

_BB = 512
_BC = 8192


def _loss_kernel(x_ref, t_ref, f_ref, out_ref):
    x = x_ref[:, :]
    norm = jnp.sqrt(jnp.sum(x * x, axis=1, keepdims=True))
    xn = x * (1.4426950408889634 / (jnp.maximum(norm, 1e-12) * _TEMP))

    t = t_ref[0, 0, :]

    n_chunks = (2 * _K) // _BC
    half = n_chunks // 2
    liota = jax.lax.broadcasted_iota(jnp.int32, (_BB, _BC), 1)

    def body(ci, carry):
        sm, sh, tm, th = carry
        chunk = f_ref[pl.ds(ci * _BC, _BC), :]
        logits = jax.lax.dot_general(
            xn, chunk,
            dimension_numbers=(((1,), (1,)), ((), ())),
            preferred_element_type=jnp.float32,
            precision=jax.lax.Precision.DEFAULT,
        )
        e = jnp.exp2(logits)
        s = jnp.sum(e, axis=1)
        is_mean = ci < half
        tloc = t - jnp.where(is_mean, ci * _BC, (ci - half) * _BC)
        contrib = jnp.sum(jnp.where(liota == tloc[:, None], logits, 0.0), axis=1)
        tm = tm + jnp.where(is_mean, contrib, 0.0)
        th = th + jnp.where(is_mean, 0.0, contrib)
        sm = jnp.where(is_mean, sm + s, sm)
        sh = jnp.where(is_mean, sh, sh + s)
        return sm, sh, tm, th

    z = jnp.zeros((_BB,), jnp.float32)
    sm, sh, tm, th = jax.lax.fori_loop(0, n_chunks, body, (z, z, z, z))

    ln2 = 0.6931471805599453
    nll = (0.5 * ln2) * ((jnp.log2(sm) - tm) + (jnp.log2(sh) - th))
    out_ref[:, 0] = nll


@jax.jit
def _run(inputs, targets, features):
    nb = _B // _BB
    t3 = targets.astype(jnp.int32).reshape(nb, 1, _BB)
    nll = pl.pallas_call(
        _loss_kernel,
        grid=(nb,),
        in_specs=[
            pl.BlockSpec((_BB, _D), lambda b: (b, 0)),
            pl.BlockSpec((1, 1, _BB), lambda b: (b, 0, 0)),
            pl.BlockSpec((2 * _K, _D), lambda b: (0, 0)),
        ],
        out_specs=pl.BlockSpec((_BB, 1), lambda b: (b, 0)),
        out_shape=jax.ShapeDtypeStruct((_B, 1), jnp.float32),
    )(inputs, t3, features)
    return jnp.mean(nll)


def kernel(inputs, targets, features):
    return _run(inputs, targets, features)


# speedup vs baseline: 6.3660x; 1.0430x over previous
import functools

import jax
import jax.numpy as jnp
from jax.experimental import pallas as pl

_B = 4096
_D = 256
_K = 8192
_TEMP = 0.05

_BB = 1024
_BC = 8192


def _loss_kernel(x_ref, t_ref, f_ref, out_ref):
    x = x_ref[:, :]
    norm = jnp.sqrt(jnp.sum(x * x, axis=1, keepdims=True))
    xn = x * (1.4426950408889634 / (jnp.maximum(norm, 1e-12) * _TEMP))

    t = t_ref[0, 0, :]

    n_chunks = (2 * _K) // _BC
    half = n_chunks // 2
    liota = jax.lax.broadcasted_iota(jnp.int32, (_BB, _BC), 1)

    def body(ci, carry):
        sm, sh, tm, th = carry
        chunk = f_ref[pl.ds(ci * _BC, _BC), :]
        logits = jax.lax.dot_general(
            xn, chunk,
            dimension_numbers=(((1,), (1,)), ((), ())),
            preferred_element_type=jnp.float32,
            precision=jax.lax.Precision.DEFAULT,
        )
        e = jnp.exp2(logits)
        s = jnp.sum(e, axis=1)
        is_mean = ci < half
        tloc = t - jnp.where(is_mean, ci * _BC, (ci - half) * _BC)
        contrib = jnp.sum(jnp.where(liota == tloc[:, None], logits, 0.0), axis=1)
        tm = tm + jnp.where(is_mean, contrib, 0.0)
        th = th + jnp.where(is_mean, 0.0, contrib)
        sm = jnp.where(is_mean, sm + s, sm)
        sh = jnp.where(is_mean, sh, sh + s)
        return sm, sh, tm, th

    z = jnp.zeros((_BB,), jnp.float32)
    sm, sh, tm, th = jax.lax.fori_loop(0, n_chunks, body, (z, z, z, z))

    ln2 = 0.6931471805599453
    nll = (0.5 * ln2) * ((jnp.log2(sm) - tm) + (jnp.log2(sh) - th))
    out_ref[:, 0] = nll


@jax.jit
def _run(inputs, targets, features):
    nb = _B // _BB
    t3 = targets.astype(jnp.int32).reshape(nb, 1, _BB)
    nll = pl.pallas_call(
        _loss_kernel,
        grid=(nb,),
        in_specs=[
            pl.BlockSpec((_BB, _D), lambda b: (b, 0)),
            pl.BlockSpec((1, 1, _BB), lambda b: (b, 0, 0)),
            pl.BlockSpec((2 * _K, _D), lambda b: (0, 0)),
        ],
        out_specs=pl.BlockSpec((_BB, 1), lambda b: (b, 0)),
        out_shape=jax.ShapeDtypeStruct((_B, 1), jnp.float32),
    )(inputs, t3, features)
    return jnp.mean(nll)


def kernel(inputs, targets, features):
    return _run(inputs, targets, features)


# speedup vs baseline: 6.9732x; 1.0954x over previous
import functools

import jax
import jax.numpy as jnp
from jax import lax
from jax.experimental import pallas as pl
from jax.experimental.pallas import tpu as pltpu
from jax.experimental.pallas import tpu_sc as plsc

_B = 4096
_D = 256
_K = 8192
_TEMP = 0.05
_LOG2E = 1.4426950408889634
_LN2 = 0.6931471805599453

_BB = 1024
_BC = 8192

_NC = 2
_NS = 16
_NW = _NC * _NS
_BPW = _B // _NW


def _lse_kernel(x_ref, f_ref, a_ref, s_ref):
    x = x_ref[:, :]
    norm = jnp.maximum(jnp.sqrt(jnp.sum(x * x, axis=1, keepdims=True)), 1e-12)
    xn = x * (_LOG2E / (norm * _TEMP))

    n_chunks = (2 * _K) // _BC
    half = n_chunks // 2

    def body(ci, carry):
        sm, sh = carry
        chunk = f_ref[pl.ds(ci * _BC, _BC), :]
        logits = jax.lax.dot_general(
            xn, chunk,
            dimension_numbers=(((1,), (1,)), ((), ())),
            preferred_element_type=jnp.float32,
            precision=jax.lax.Precision.DEFAULT,
        )
        s = jnp.sum(jnp.exp2(logits), axis=1)
        is_mean = ci < half
        sm = jnp.where(is_mean, sm + s, sm)
        sh = jnp.where(is_mean, sh, sh + s)
        return sm, sh

    z = jnp.zeros((_BB,), jnp.float32)
    sm, sh = jax.lax.fori_loop(0, n_chunks, body, (z, z))

    a_ref[:, 0] = (0.5 * _LN2) * (jnp.log2(sm) + jnp.log2(sh))
    s_ref[:, 0] = 0.5 / (norm[:, 0] * _TEMP)


def _target_dot_kernel(x_hbm, t_hbm, f_hbm, dm_hbm, dh_hbm,
                       idx_v, idx2_v, x_v, rm_v, rh_v, dm_v, dh_v, sem):
    wid = lax.axis_index("s") * _NC + lax.axis_index("c")
    base = wid * _BPW

    pltpu.sync_copy(t_hbm.at[pl.ds(base, _BPW)], idx_v)
    pltpu.sync_copy(x_hbm.at[pl.ds(base, _BPW), :], x_v)
    for i in range(_BPW // 16):
        idx2_v[pl.ds(i * 16, 16)] = idx_v[pl.ds(i * 16, 16)] + _K
    pltpu.async_copy(f_hbm.at[idx_v], rm_v, sem).wait()
    pltpu.async_copy(f_hbm.at[idx2_v], rh_v, sem).wait()

    def row(r, c):
        zm = jnp.zeros((16,), jnp.float32)
        zh = jnp.zeros((16,), jnp.float32)
        for j in range(_D // 16):
            xx = x_v[r, pl.ds(j * 16, 16)]
            zm = zm + xx * rm_v[r, pl.ds(j * 16, 16)]
            zh = zh + xx * rh_v[r, pl.ds(j * 16, 16)]
        dm_v[pl.ds(r * 16, 16)] = zm
        dh_v[pl.ds(r * 16, 16)] = zh
        return c

    lax.fori_loop(0, _BPW, row, 0)
    pltpu.sync_copy(dm_v, dm_hbm.at[wid])
    pltpu.sync_copy(dh_v, dh_hbm.at[wid])


_sc_target_dots = functools.partial(
    pl.kernel,
    out_type=[
        jax.ShapeDtypeStruct((_NW, _BPW * 16), jnp.float32),
        jax.ShapeDtypeStruct((_NW, _BPW * 16), jnp.float32),
    ],
    mesh=plsc.VectorSubcoreMesh(core_axis_name="c", subcore_axis_name="s"),
    scratch_types=[
        pltpu.VMEM((_BPW,), jnp.int32),
        pltpu.VMEM((_BPW,), jnp.int32),
        pltpu.VMEM((_BPW, _D), jnp.float32),
        pltpu.VMEM((_BPW, _D), jnp.float32),
        pltpu.VMEM((_BPW, _D), jnp.float32),
        pltpu.VMEM((_BPW * 16,), jnp.float32),
        pltpu.VMEM((_BPW * 16,), jnp.float32),
        pltpu.SemaphoreType.DMA,
    ],
)(_target_dot_kernel)


@jax.jit
def _run(inputs, targets, features):
    t32 = targets.astype(jnp.int32)
    dm, dh = _sc_target_dots(inputs, t32, features)

    nb = _B // _BB
    a, s = pl.pallas_call(
        _lse_kernel,
        grid=(nb,),
        in_specs=[
            pl.BlockSpec((_BB, _D), lambda b: (b, 0)),
            pl.BlockSpec((2 * _K, _D), lambda b: (0, 0)),
        ],
        out_specs=[
            pl.BlockSpec((_BB, 1), lambda b: (b, 0)),
            pl.BlockSpec((_BB, 1), lambda b: (b, 0)),
        ],
        out_shape=[
            jax.ShapeDtypeStruct((_B, 1), jnp.float32),
            jax.ShapeDtypeStruct((_B, 1), jnp.float32),
        ],
    )(inputs, features)

    tgt = jnp.sum((dm + dh).reshape(_B, 16), axis=1)
    return jnp.mean(a[:, 0] - s[:, 0] * tgt)


def kernel(inputs, targets, features):
    return _run(inputs, targets, features)
